# baseline (speedup 1.0000x reference)
import jax
import jax.numpy as jnp
import numpy as np
from jax.experimental import pallas as pl
from jax.experimental.pallas import tpu as pltpu

_ANCHORS_ALL = np.array(
    [[10, 13], [16, 30], [33, 23], [30, 61], [62, 45], [59, 119],
     [116, 90], [156, 198], [373, 326]], dtype=np.float32)
_ANCHORS_MASK = np.array([0, 1, 2], dtype=np.int32)
_NUM_CLASSES = 80
_C = 5 + _NUM_CLASSES
_NA = 3

_NB, _NCH, _NH, _NW = 16, 255, 64, 64
_HBLK = 64

_AW = _ANCHORS_ALL[_ANCHORS_MASK, 0]
_AH = _ANCHORS_ALL[_ANCHORS_MASK, 1]


def _decode_kernel(dim_ref, x_ref, o_ref):
    j = pl.program_id(1)
    v = x_ref[0]
    hw = _HBLK * _NW
    v2d = v.reshape(_NCH, hw)
    eye_c = jnp.eye(_C, dtype=jnp.float32)
    dn = (((0,), (0,)), ((), ()))

    n = jax.lax.broadcasted_iota(jnp.int32, (hw, 1), 0)
    c = jax.lax.broadcasted_iota(jnp.int32, (1, _C), 1)
    w_f = (n % _NW).astype(jnp.float32)
    h_f = (n // _NW + j * _HBLK).astype(jnp.float32)
    is_wh = jnp.logical_or(c == 2, c == 3)
    stride = dim_ref[1].astype(jnp.float32) / _NH
    sp = jnp.where(c < 2, stride, 1.0)
    rp = jnp.where(c < 2, jnp.where(c == 0, w_f, h_f) * stride, 0.0)
    sgn_eye = eye_c * jnp.where(is_wh, 1.0, -1.0)

    for a in range(_NA):
        slab = jax.lax.dot_general(
            v2d[_C * a:_C * (a + 1), :], sgn_eye, dn,
            precision=jax.lax.Precision.HIGHEST,
            preferred_element_type=jnp.float32)
        e = jnp.exp(slab)
        sig = 1.0 / (1.0 + e)
        mul = jnp.where(c == 2, _AW[a], _AH[a])
        res = jnp.where(is_wh, e * mul, sig * sp + rp)
        o_ref[0, pl.Slice(a, hw, _NA), :] = res


def kernel(x, img_dim):
    nB, nCh, nH, nW = x.shape
    grid = (nB, nH // _HBLK)
    out = pl.pallas_call(
        _decode_kernel,
        grid_spec=pltpu.PrefetchScalarGridSpec(
            num_scalar_prefetch=1,
            grid=grid,
            in_specs=[
                pl.BlockSpec((1, nCh, _HBLK, nW), lambda b, j, dim: (b, 0, j, 0)),
            ],
            out_specs=pl.BlockSpec((1, _HBLK * nW * _NA, _C),
                                   lambda b, j, dim: (b, j, 0)),
        ),
        out_shape=jax.ShapeDtypeStruct((nB, nH * nW * _NA, _C), x.dtype),
        compiler_params=pltpu.CompilerParams(
            dimension_semantics=("parallel", "parallel")),
    )(img_dim, x)
    return out


# speedup vs baseline: 1.1636x; 1.1636x over previous
import jax
import jax.numpy as jnp
import numpy as np
from jax.experimental import pallas as pl
from jax.experimental.pallas import tpu as pltpu

_ANCHORS_ALL = np.array(
    [[10, 13], [16, 30], [33, 23], [30, 61], [62, 45], [59, 119],
     [116, 90], [156, 198], [373, 326]], dtype=np.float32)
_ANCHORS_MASK = np.array([0, 1, 2], dtype=np.int32)
_NUM_CLASSES = 80
_C = 5 + _NUM_CLASSES
_NA = 3

_NB, _NCH, _NH, _NW = 16, 255, 64, 64
_HBLK = 32

_AW = _ANCHORS_ALL[_ANCHORS_MASK, 0]
_AH = _ANCHORS_ALL[_ANCHORS_MASK, 1]


def _decode_kernel(dim_ref, x_ref, o_ref):
    j = pl.program_id(1)
    v = x_ref[0]
    hw = _HBLK * _NW
    v2d = v.reshape(_NCH, hw)
    eye_c = jnp.eye(_C, dtype=jnp.float32)
    dn = (((0,), (0,)), ((), ()))

    n = jax.lax.broadcasted_iota(jnp.int32, (hw, 1), 0)
    c = jax.lax.broadcasted_iota(jnp.int32, (1, _C), 1)
    w_f = (n % _NW).astype(jnp.float32)
    h_f = (n // _NW + j * _HBLK).astype(jnp.float32)
    is_wh = jnp.logical_or(c == 2, c == 3)
    stride = dim_ref[1].astype(jnp.float32) / _NH
    sp = jnp.where(c < 2, stride, 1.0)
    rp = jnp.where(c < 2, jnp.where(c == 0, w_f, h_f) * stride, 0.0)
    sgn_eye = eye_c * jnp.where(is_wh, 1.0, -1.0)

    for a in range(_NA):
        slab = jax.lax.dot_general(
            v2d[_C * a:_C * (a + 1), :], sgn_eye, dn,
            preferred_element_type=jnp.float32)
        e = jnp.exp(slab)
        sig = 1.0 / (1.0 + e)
        mul = jnp.where(c == 2, _AW[a], _AH[a])
        res = jnp.where(is_wh, e * mul, sig * sp + rp)
        o_ref[0, pl.Slice(a, hw, _NA), :] = res


def kernel(x, img_dim):
    nB, nCh, nH, nW = x.shape
    grid = (nB, nH // _HBLK)
    out = pl.pallas_call(
        _decode_kernel,
        grid_spec=pltpu.PrefetchScalarGridSpec(
            num_scalar_prefetch=1,
            grid=grid,
            in_specs=[
                pl.BlockSpec((1, nCh, _HBLK, nW), lambda b, j, dim: (b, 0, j, 0)),
            ],
            out_specs=pl.BlockSpec((1, _HBLK * nW * _NA, _C),
                                   lambda b, j, dim: (b, j, 0)),
        ),
        out_shape=jax.ShapeDtypeStruct((nB, nH * nW * _NA, _C), x.dtype),
        compiler_params=pltpu.CompilerParams(
            dimension_semantics=("parallel", "parallel")),
    )(img_dim, x)
    return out
